# Initial kernel scaffold; baseline (speedup 1.0000x reference)
import jax
import jax.numpy as jnp
from jax.experimental import pallas as pl


def kernel(x, edge_index):
    raise NotImplementedError("write your pallas kernel here")


# speedup vs baseline: 3.0093x; 3.0093x over previous
import jax
import jax.numpy as jnp
from jax import lax
from jax.experimental import pallas as pl
from jax.experimental.pallas import tpu as pltpu
from jax.experimental.pallas import tpu_sc as plsc

N = 10000
E = 160000
D = 256
HALF = 128
NLAYERS = 3

NC = 2
NS = 16
NW = NC * NS
L = 16

NROWS = 10240
CK = 128
E_PAD = 163840
NCHUNK = E_PAD // (NW * CK)
RPT = NROWS // NS
NZ = RPT // 64


def _gcn_body(x2, srcA, srcB, dst3, out, h1, h2, accum, counts,
              src_v, dst_v, rows, ones_t, acc_t, rec_t, gsem):
    c = lax.axis_index("c")
    s = lax.axis_index("s")

    @pl.loop(0, 64)
    def _fill64(i):
        rec_t[i, :] = jnp.zeros((L,), jnp.float32)

    @pl.loop(0, CK)
    def _fill_ones(i):
        ones_t[i, :] = jnp.ones((L,), jnp.float32)

    @pl.loop(0, NZ)
    def _zero_counts(i):
        pltpu.sync_copy(rec_t, counts.at[pl.ds(s * RPT + i * 64, 64)])

    plsc.subcore_barrier()

    for half in range(2):
        pltpu.sync_copy(dst3.at[2 * s + half], dst_v)

        @pl.loop(0, NCHUNK)
        def _count(j):
            pltpu.sync_copy(ones_t, counts.at[dst_v.at[j]], add=True)

    plsc.subcore_barrier()

    @pl.loop(0, NZ)
    def _recip(i):
        r0 = s * RPT + i * 64
        pltpu.sync_copy(counts.at[pl.ds(r0, 64)], rec_t)

        @pl.loop(0, 64)
        def _rrow(r):
            rec_t[r, :] = 1.0 / jnp.maximum(rec_t[r, :], 1.0)

        pltpu.sync_copy(rec_t, counts.at[pl.ds(r0, 64)])

    plsc.subcore_barrier()

    for layer in range(NLAYERS):
        @pl.loop(0, 64)
        def _zero_acc_t(i):
            for l in range(HALF // L):
                acc_t[i, pl.ds(l * L, L)] = jnp.zeros((L,), jnp.float32)

        @pl.loop(0, NZ)
        def _zero_acc(i):
            pltpu.sync_copy(acc_t, accum.at[pl.ds(s * RPT + i * 64, 64)])

        plsc.subcore_barrier()

        gsrc = (x2, h1, h2)[layer]
        for half in range(2):
            slab = 2 * s + half
            pltpu.sync_copy((srcA if layer == 0 else srcB).at[c, slab], src_v)
            pltpu.sync_copy(dst3.at[slab], dst_v)

            @pl.loop(0, NCHUNK)
            def _edges(j):
                pltpu.async_copy(gsrc.at[src_v.at[j]], rows, gsem).wait()
                pltpu.sync_copy(rows, accum.at[dst_v.at[j]], add=True)

        plsc.subcore_barrier()

        @pl.loop(0, NZ)
        def _norm(i):
            r0 = s * RPT + i * 64
            pltpu.sync_copy(accum.at[pl.ds(r0, 64)], acc_t)
            pltpu.sync_copy(counts.at[pl.ds(r0, 64)], rec_t)

            @pl.loop(0, 64)
            def _nrow(r):
                rv = rec_t[r, :]
                for l in range(HALF // L):
                    acc_t[r, pl.ds(l * L, L)] = acc_t[r, pl.ds(l * L, L)] * rv

            if layer < NLAYERS - 1:
                hdst = (h1, h2)[layer]
                pltpu.sync_copy(acc_t, hdst.at[pl.ds(c * NROWS + r0, 64)])
            else:
                @pl.when(r0 + 64 <= N)
                def _full():
                    pltpu.sync_copy(
                        acc_t, out.at[pl.ds(r0, 64), pl.ds(c * HALF, HALF)])

                @pl.when(jnp.logical_and(r0 < N, r0 + 64 > N))
                def _part():
                    pltpu.sync_copy(
                        acc_t.at[pl.ds(0, N % 64)],
                        out.at[pl.ds(r0, N % 64), pl.ds(c * HALF, HALF)])

        plsc.subcore_barrier()


def kernel(x, edge_index):
    src = edge_index[0].astype(jnp.int32)
    dst = edge_index[1].astype(jnp.int32)
    npad = E_PAD - E
    src = jnp.concatenate([src, jnp.zeros((npad,), jnp.int32)])
    dst = jnp.concatenate([dst, jnp.full((npad,), N, jnp.int32)])

    srcA = jnp.stack([2 * src, 2 * src + 1]).reshape(NC, NW, NCHUNK, CK)
    srcB = jnp.stack([src, NROWS + src]).reshape(NC, NW, NCHUNK, CK)
    dst3 = dst.reshape(NW, NCHUNK, CK)
    x2 = x.reshape(2 * N, HALF)

    mesh = plsc.VectorSubcoreMesh(core_axis_name="c", subcore_axis_name="s")
    f = pl.kernel(
        _gcn_body,
        out_type=jax.ShapeDtypeStruct((N, D), jnp.float32),
        mesh=mesh,
        compiler_params=pltpu.CompilerParams(use_tc_tiling_on_sc=False),
        scratch_types=[
            pltpu.HBM((NC * NROWS, HALF), jnp.float32),
            pltpu.HBM((NC * NROWS, HALF), jnp.float32),
            pltpu.VMEM_SHARED((NROWS, HALF), jnp.float32),
            pltpu.VMEM_SHARED((NROWS, L), jnp.float32),
            pltpu.VMEM((NCHUNK, CK), jnp.int32),
            pltpu.VMEM((NCHUNK, CK), jnp.int32),
            pltpu.VMEM((CK, HALF), jnp.float32),
            pltpu.VMEM((CK, L), jnp.float32),
            pltpu.VMEM((64, HALF), jnp.float32),
            pltpu.VMEM((64, L), jnp.float32),
            pltpu.SemaphoreType.DMA,
        ],
    )
    return f(x2, srcA, srcB, dst3)


# speedup vs baseline: 3.2398x; 1.0766x over previous
import jax
import jax.numpy as jnp
from jax import lax
from jax.experimental import pallas as pl
from jax.experimental.pallas import tpu as pltpu
from jax.experimental.pallas import tpu_sc as plsc

N = 10000
E = 160000
D = 256
HALF = 128
NLAYERS = 3

NC = 2
NS = 16
NW = NC * NS
L = 16

NROWS = 10240
CK = 64
E_PAD = 163840
NCHUNK = E_PAD // (NW * CK)
RPT = NROWS // NS
NZ = RPT // 64


def _gcn_body(x2, srcA, srcB, dst3, out, h1, h2, accum, counts,
              src_v, dst_v, rows0, rows1, ones_t, acc_t, rec_t, gsem0, gsem1):
    c = lax.axis_index("c")
    s = lax.axis_index("s")

    @pl.loop(0, 64)
    def _fill64(i):
        rec_t[i, :] = jnp.zeros((L,), jnp.float32)

    @pl.loop(0, CK)
    def _fill_ones(i):
        ones_t[i, :] = jnp.ones((L,), jnp.float32)

    def _gather(gsrc, j, buf, sem):
        return pltpu.make_async_copy(gsrc.at[src_v.at[j]], buf, sem)

    @pl.loop(0, NZ)
    def _zero_counts(i):
        pltpu.sync_copy(rec_t, counts.at[pl.ds(s * RPT + i * 64, 64)])

    plsc.subcore_barrier()

    for half in range(2):
        pltpu.sync_copy(dst3.at[2 * s + half], dst_v)

        @pl.loop(0, NCHUNK)
        def _count(j):
            pltpu.sync_copy(ones_t, counts.at[dst_v.at[j]], add=True)

    plsc.subcore_barrier()

    @pl.loop(0, NZ)
    def _recip(i):
        r0 = s * RPT + i * 64
        pltpu.sync_copy(counts.at[pl.ds(r0, 64)], rec_t)

        @pl.loop(0, 64)
        def _rrow(r):
            rec_t[r, :] = 1.0 / jnp.maximum(rec_t[r, :], 1.0)

        pltpu.sync_copy(rec_t, counts.at[pl.ds(r0, 64)])

    plsc.subcore_barrier()

    for layer in range(NLAYERS):
        @pl.loop(0, 64)
        def _zero_acc_t(i):
            for l in range(HALF // L):
                acc_t[i, pl.ds(l * L, L)] = jnp.zeros((L,), jnp.float32)

        @pl.loop(0, NZ)
        def _zero_acc(i):
            pltpu.sync_copy(acc_t, accum.at[pl.ds(s * RPT + i * 64, 64)])

        plsc.subcore_barrier()

        gsrc = (x2, h1, h2)[layer]
        for half in range(2):
            slab = 2 * s + half
            pltpu.sync_copy((srcA if layer == 0 else srcB).at[c, slab], src_v)
            pltpu.sync_copy(dst3.at[slab], dst_v)

            _gather(gsrc, 0, rows0, gsem0).start()

            @pl.loop(0, NCHUNK, step=2)
            def _edges(j):
                _gather(gsrc, j, rows0, gsem0).wait()
                _gather(gsrc, j + 1, rows1, gsem1).start()
                pltpu.sync_copy(rows0, accum.at[dst_v.at[j]], add=True)
                _gather(gsrc, j + 1, rows1, gsem1).wait()

                @pl.when(j + 2 < NCHUNK)
                def _next():
                    _gather(gsrc, j + 2, rows0, gsem0).start()

                pltpu.sync_copy(rows1, accum.at[dst_v.at[j + 1]], add=True)

        plsc.subcore_barrier()

        @pl.loop(0, NZ)
        def _norm(i):
            r0 = s * RPT + i * 64
            pltpu.sync_copy(accum.at[pl.ds(r0, 64)], acc_t)
            pltpu.sync_copy(counts.at[pl.ds(r0, 64)], rec_t)

            @pl.loop(0, 64)
            def _nrow(r):
                rv = rec_t[r, :]
                for l in range(HALF // L):
                    acc_t[r, pl.ds(l * L, L)] = acc_t[r, pl.ds(l * L, L)] * rv

            if layer < NLAYERS - 1:
                hdst = (h1, h2)[layer]
                pltpu.sync_copy(acc_t, hdst.at[pl.ds(c * NROWS + r0, 64)])
            else:
                @pl.when(r0 + 64 <= N)
                def _full():
                    pltpu.sync_copy(
                        acc_t, out.at[pl.ds(r0, 64), pl.ds(c * HALF, HALF)])

                @pl.when(jnp.logical_and(r0 < N, r0 + 64 > N))
                def _part():
                    pltpu.sync_copy(
                        acc_t.at[pl.ds(0, N % 64)],
                        out.at[pl.ds(r0, N % 64), pl.ds(c * HALF, HALF)])

        plsc.subcore_barrier()


def kernel(x, edge_index):
    src = edge_index[0].astype(jnp.int32)
    dst = edge_index[1].astype(jnp.int32)
    npad = E_PAD - E
    src = jnp.concatenate([src, jnp.zeros((npad,), jnp.int32)])
    dst = jnp.concatenate([dst, jnp.full((npad,), N, jnp.int32)])

    srcA = jnp.stack([2 * src, 2 * src + 1]).reshape(NC, NW, NCHUNK, CK)
    srcB = jnp.stack([src, NROWS + src]).reshape(NC, NW, NCHUNK, CK)
    dst3 = dst.reshape(NW, NCHUNK, CK)
    x2 = x.reshape(2 * N, HALF)

    mesh = plsc.VectorSubcoreMesh(core_axis_name="c", subcore_axis_name="s")
    f = pl.kernel(
        _gcn_body,
        out_type=jax.ShapeDtypeStruct((N, D), jnp.float32),
        mesh=mesh,
        compiler_params=pltpu.CompilerParams(use_tc_tiling_on_sc=False),
        scratch_types=[
            pltpu.HBM((NC * NROWS, HALF), jnp.float32),
            pltpu.HBM((NC * NROWS, HALF), jnp.float32),
            pltpu.VMEM_SHARED((NROWS, HALF), jnp.float32),
            pltpu.VMEM_SHARED((NROWS, L), jnp.float32),
            pltpu.VMEM((NCHUNK, CK), jnp.int32),
            pltpu.VMEM((NCHUNK, CK), jnp.int32),
            pltpu.VMEM((CK, HALF), jnp.float32),
            pltpu.VMEM((CK, HALF), jnp.float32),
            pltpu.VMEM((CK, L), jnp.float32),
            pltpu.VMEM((64, HALF), jnp.float32),
            pltpu.VMEM((64, L), jnp.float32),
            pltpu.SemaphoreType.DMA,
            pltpu.SemaphoreType.DMA,
        ],
    )
    return f(x2, srcA, srcB, dst3)
